# baseline (speedup 1.0000x reference)
import functools

import jax
import jax.numpy as jnp
from jax import lax
from jax.experimental import pallas as pl
from jax.experimental.pallas import tpu as pltpu
from jax.experimental.pallas import tpu_sc as plsc

KEEP = 2048
G = 1
SC_NC = 2
SC_NS = 16
LANES = 16
ROWS_PER_GATHER = 128


def _tc_body(x_ref, s_ref, ti_ref, xt_ref):
    xb = x_ref[...]
    g, C, H, W = xb.shape
    s = jnp.sum(jnp.abs(xb), axis=1)
    sb = lax.bitcast_convert_type(s, jnp.int32)

    def vstep(i, t):
        cand = t + ((1 << 30) >> i)
        cnt = jnp.sum((sb >= cand[:, None, None]).astype(jnp.int32), axis=(1, 2))
        return jnp.where(cnt >= KEEP, cand, t)

    t = lax.fori_loop(0, 31, vstep, jnp.zeros((g,), jnp.int32))

    gt = sb > t[:, None, None]
    eq = sb == t[:, None, None]
    need = KEEP - jnp.sum(gt.astype(jnp.int32), axis=(1, 2))
    row = lax.broadcasted_iota(jnp.int32, (H, W), 0)
    col = lax.broadcasted_iota(jnp.int32, (H, W), 1)
    nidx = (row * W + col)[None]

    def istep(i, lo):
        cand = lo + ((1 << 13) >> i)
        cnt = jnp.sum((eq & (nidx <= cand[:, None, None])).astype(jnp.int32),
                      axis=(1, 2))
        return jnp.where(cnt < need, cand, lo)

    lo = lax.fori_loop(0, 14, istep, jnp.full((g,), -1, jnp.int32))

    s_ref[...] = sb
    tvec = jnp.broadcast_to(t[:, None, None], (g, 1, LANES))
    cvec = jnp.broadcast_to((lo + 1)[:, None, None], (g, 1, LANES))
    ti_ref[...] = jnp.concatenate([tvec, cvec], axis=1)
    xt_ref[:, :, :, 0:C] = jnp.transpose(xb, (0, 2, 3, 1))


def _sc_pool(scores, tinfo, xt2d, B, C, N):
    mesh = plsc.VectorSubcoreMesh(core_axis_name="c", subcore_axis_name="s")
    n_chunks = KEEP // ROWS_PER_GATHER
    cg = C // LANES

    @functools.partial(
        pl.kernel,
        mesh=mesh,
        out_type=jax.ShapeDtypeStruct((B, cg, LANES), jnp.float32),
        scratch_types=[
            pltpu.VMEM((N,), jnp.int32),
            pltpu.VMEM((2, LANES), jnp.int32),
            pltpu.VMEM((KEEP + 128,), jnp.int32),
            pltpu.VMEM((n_chunks, ROWS_PER_GATHER), jnp.int32),
            pltpu.VMEM((ROWS_PER_GATHER, 128), jnp.float32),
            pltpu.VMEM((cg, LANES), jnp.float32),
            pltpu.SemaphoreType.DMA,
        ],
    )
    def body(scores_hbm, tinfo_hbm, xt_hbm, out_hbm,
             sc_v, ti_v, idxf_v, idx2_v, buf_v, outrow_v, sem):
        b = lax.axis_index("s") * SC_NC + lax.axis_index("c")
        pltpu.sync_copy(scores_hbm.at[b], sc_v)
        pltpu.sync_copy(tinfo_hbm.at[b], ti_v)
        tspl = ti_v[0]
        cutspl = ti_v[1]
        lane = lax.iota(jnp.int32, LANES)
        base = b * N

        def take16(v, idx):
            dnums = lax.GatherDimensionNumbers(
                offset_dims=(), collapsed_slice_dims=(0,),
                start_index_map=(0,))
            return lax.gather(
                v, idx[:, None], dnums, (1,),
                mode=lax.GatherScatterMode.PROMISE_IN_BOUNDS)

        ones = jnp.full((LANES,), 1, jnp.int32)
        zeros_i = jnp.zeros((LANES,), jnp.int32)
        def cstep(i, off):
            v = sc_v[pl.ds(i * LANES, LANES)]
            idxs = lane + i * LANES
            m = (v > tspl) | ((v == tspl) & (idxs <= cutspl))
            mi = jnp.where(m, ones, zeros_i)
            csum = mi
            for sh in (1, 2, 4, 8):
                shifted = take16(csum, jnp.maximum(lane - sh, 0))
                csum = csum + jnp.where(lane >= sh, shifted, zeros_i)
            tgt = lane + 1
            pos = zeros_i
            for step in (8, 4, 2, 1):
                cand = pos + step
                cval = take16(csum, cand - 1)
                pos = jnp.where(cval < tgt, cand, pos)
            vals = take16(idxs, pos) + base
            idxf_v[pl.ds(off, LANES)] = vals
            return off + csum[LANES - 1]

        lax.fori_loop(0, N // LANES, cstep, jnp.int32(0))

        for r in range(n_chunks):
            for k in range(ROWS_PER_GATHER // LANES):
                idx2_v[r, pl.ds(k * LANES, LANES)] = (
                    idxf_v[pl.ds(r * ROWS_PER_GATHER + k * LANES, LANES)])

        zeros = jnp.zeros((LANES,), jnp.float32)

        def gstep(q, acc):
            pltpu.async_copy(xt_hbm.at[idx2_v.at[q]], buf_v, sem).wait()

            def rstep(j, a):
                return tuple(a[k] + buf_v[j, pl.ds(k * LANES, LANES)]
                             for k in range(cg))

            return lax.fori_loop(0, ROWS_PER_GATHER, rstep, acc)

        acc = lax.fori_loop(0, n_chunks, gstep, (zeros,) * cg)
        for k in range(cg):
            outrow_v[k] = acc[k] * (1.0 / KEEP)
        pltpu.sync_copy(outrow_v, out_hbm.at[b])

    return body(scores, tinfo, xt2d)


def kernel(x):
    B, C, H, W = x.shape
    N = H * W
    sbits, tinfo, xt = pl.pallas_call(
        _tc_body,
        grid=(B // G,),
        in_specs=[pl.BlockSpec((G, C, H, W), lambda b: (b, 0, 0, 0))],
        out_specs=[
            pl.BlockSpec((G, H, W), lambda b: (b, 0, 0)),
            pl.BlockSpec((G, 2, LANES), lambda b: (b, 0, 0)),
            pl.BlockSpec((G, H, W, 128), lambda b: (b, 0, 0, 0)),
        ],
        out_shape=[
            jax.ShapeDtypeStruct((B, H, W), jnp.int32),
            jax.ShapeDtypeStruct((B, 2, LANES), jnp.int32),
            jax.ShapeDtypeStruct((B, H, W, 128), jnp.float32),
        ],
    )(x)
    scores = sbits.reshape(B, N)
    xt2d = xt.reshape(B * N, 128)
    out = _sc_pool(scores, tinfo, xt2d, B, C, N)
    return out.reshape(B, C)


# speedup vs baseline: 1.5089x; 1.5089x over previous
import functools

import jax
import jax.numpy as jnp
from jax import lax
from jax.experimental import pallas as pl
from jax.experimental.pallas import tpu as pltpu
from jax.experimental.pallas import tpu_sc as plsc

KEEP = 2048
G = 1
SC_NC = 2
SC_NS = 16
LANES = 16
ROWS_PER_GATHER = 128


def _tc_stream_body(x_ref, s_ref, xt_ref):
    xb = x_ref[...]
    g, C, H, W = xb.shape
    s = jnp.sum(jnp.abs(xb), axis=1)
    s_ref[...] = lax.bitcast_convert_type(s, jnp.int32)
    xt_ref[:, :, :, 0:C] = jnp.transpose(xb, (0, 2, 3, 1))


def _tc_search_body(s_ref, ti_ref):
    sb = s_ref[...]
    B, H, W = sb.shape

    def vstep(i, t):
        cand = t + ((1 << 30) >> i)
        cnt = jnp.sum((sb >= cand[:, None, None]).astype(jnp.int32), axis=(1, 2))
        return jnp.where(cnt >= KEEP, cand, t)

    t = lax.fori_loop(0, 31, vstep, jnp.zeros((B,), jnp.int32))

    gt = sb > t[:, None, None]
    eq = sb == t[:, None, None]
    need = KEEP - jnp.sum(gt.astype(jnp.int32), axis=(1, 2))
    row = lax.broadcasted_iota(jnp.int32, (H, W), 0)
    col = lax.broadcasted_iota(jnp.int32, (H, W), 1)
    nidx = (row * W + col)[None]

    def istep(i, lo):
        cand = lo + ((1 << 13) >> i)
        cnt = jnp.sum((eq & (nidx <= cand[:, None, None])).astype(jnp.int32),
                      axis=(1, 2))
        return jnp.where(cnt < need, cand, lo)

    lo = lax.fori_loop(0, 14, istep, jnp.full((B,), -1, jnp.int32))

    tvec = jnp.broadcast_to(t[:, None, None], (B, 1, LANES))
    cvec = jnp.broadcast_to((lo + 1)[:, None, None], (B, 1, LANES))
    ti_ref[...] = jnp.concatenate([tvec, cvec], axis=1)


def _sc_pool(scores, tinfo, xt2d, B, C, N):
    mesh = plsc.VectorSubcoreMesh(core_axis_name="c", subcore_axis_name="s")
    n_chunks = KEEP // ROWS_PER_GATHER
    cg = C // LANES

    @functools.partial(
        pl.kernel,
        mesh=mesh,
        out_type=jax.ShapeDtypeStruct((B, cg, LANES), jnp.float32),
        scratch_types=[
            pltpu.VMEM((N,), jnp.int32),
            pltpu.VMEM((2, LANES), jnp.int32),
            pltpu.VMEM((KEEP + 128,), jnp.int32),
            pltpu.VMEM((n_chunks, ROWS_PER_GATHER), jnp.int32),
            pltpu.VMEM((ROWS_PER_GATHER, 128), jnp.float32),
            pltpu.VMEM((cg, LANES), jnp.float32),
            pltpu.SemaphoreType.DMA,
        ],
    )
    def body(scores_hbm, tinfo_hbm, xt_hbm, out_hbm,
             sc_v, ti_v, idxf_v, idx2_v, buf_v, outrow_v, sem):
        b = lax.axis_index("s") * SC_NC + lax.axis_index("c")
        pltpu.sync_copy(scores_hbm.at[b], sc_v)
        pltpu.sync_copy(tinfo_hbm.at[b], ti_v)
        tspl = ti_v[0]
        cutspl = ti_v[1]
        lane = lax.iota(jnp.int32, LANES)
        base = b * N

        def take16(v, idx):
            dnums = lax.GatherDimensionNumbers(
                offset_dims=(), collapsed_slice_dims=(0,),
                start_index_map=(0,))
            return lax.gather(
                v, idx[:, None], dnums, (1,),
                mode=lax.GatherScatterMode.PROMISE_IN_BOUNDS)

        ones = jnp.full((LANES,), 1, jnp.int32)
        zeros_i = jnp.zeros((LANES,), jnp.int32)
        def cstep(i, off):
            v = sc_v[pl.ds(i * LANES, LANES)]
            idxs = lane + i * LANES
            m = (v > tspl) | ((v == tspl) & (idxs <= cutspl))
            mi = jnp.where(m, ones, zeros_i)
            csum = mi
            for sh in (1, 2, 4, 8):
                shifted = take16(csum, jnp.maximum(lane - sh, 0))
                csum = csum + jnp.where(lane >= sh, shifted, zeros_i)
            tgt = lane + 1
            pos = zeros_i
            for step in (8, 4, 2, 1):
                cand = pos + step
                cval = take16(csum, cand - 1)
                pos = jnp.where(cval < tgt, cand, pos)
            vals = take16(idxs, pos) + base
            idxf_v[pl.ds(off, LANES)] = vals
            return off + csum[LANES - 1]

        lax.fori_loop(0, N // LANES, cstep, jnp.int32(0))

        for r in range(n_chunks):
            for k in range(ROWS_PER_GATHER // LANES):
                idx2_v[r, pl.ds(k * LANES, LANES)] = (
                    idxf_v[pl.ds(r * ROWS_PER_GATHER + k * LANES, LANES)])

        zeros = jnp.zeros((LANES,), jnp.float32)

        def gstep(q, acc):
            pltpu.async_copy(xt_hbm.at[idx2_v.at[q]], buf_v, sem).wait()

            def rstep(j, a):
                return tuple(a[k] + buf_v[j, pl.ds(k * LANES, LANES)]
                             for k in range(cg))

            return lax.fori_loop(0, ROWS_PER_GATHER, rstep, acc)

        acc = lax.fori_loop(0, n_chunks, gstep, (zeros,) * cg)
        for k in range(cg):
            outrow_v[k] = acc[k] * (1.0 / KEEP)
        pltpu.sync_copy(outrow_v, out_hbm.at[b])

    return body(scores, tinfo, xt2d)


def kernel(x):
    B, C, H, W = x.shape
    N = H * W
    sbits, xt = pl.pallas_call(
        _tc_stream_body,
        grid=(B // G,),
        in_specs=[pl.BlockSpec((G, C, H, W), lambda b: (b, 0, 0, 0))],
        out_specs=[
            pl.BlockSpec((G, H, W), lambda b: (b, 0, 0)),
            pl.BlockSpec((G, H, W, 128), lambda b: (b, 0, 0, 0)),
        ],
        out_shape=[
            jax.ShapeDtypeStruct((B, H, W), jnp.int32),
            jax.ShapeDtypeStruct((B, H, W, 128), jnp.float32),
        ],
    )(x)
    tinfo = pl.pallas_call(
        _tc_search_body,
        grid=(1,),
        in_specs=[pl.BlockSpec((B, H, W), lambda _: (0, 0, 0))],
        out_specs=pl.BlockSpec((B, 2, LANES), lambda _: (0, 0, 0)),
        out_shape=jax.ShapeDtypeStruct((B, 2, LANES), jnp.int32),
    )(sbits)
    scores = sbits.reshape(B, N)
    xt2d = xt.reshape(B * N, 128)
    out = _sc_pool(scores, tinfo, xt2d, B, C, N)
    return out.reshape(B, C)


# speedup vs baseline: 1.5695x; 1.0401x over previous
import functools

import jax
import jax.numpy as jnp
from jax import lax
from jax.experimental import pallas as pl
from jax.experimental.pallas import tpu as pltpu
from jax.experimental.pallas import tpu_sc as plsc

KEEP = 2048
G = 1
SC_NC = 2
SC_NS = 16
LANES = 16
ROWS_PER_GATHER = 128


def _tc_stream_body(x_ref, s_ref, xt_ref):
    xb = x_ref[...]
    g, C, H, W = xb.shape
    s = jnp.sum(jnp.abs(xb), axis=1)
    s_ref[...] = lax.bitcast_convert_type(s, jnp.int32)
    xt_ref[:, :, :, 0:C] = jnp.transpose(xb, (0, 2, 3, 1))


def _tc_search_body(s_ref, ti_ref):
    sb = s_ref[...]
    B, H, W = sb.shape

    def vstep(i, t):
        cand = t + ((1 << 30) >> i)
        cnt = jnp.sum((sb >= cand[:, None, None]).astype(jnp.int32), axis=(1, 2))
        return jnp.where(cnt >= KEEP, cand, t)

    t = lax.fori_loop(0, 31, vstep, jnp.zeros((B,), jnp.int32))

    gt = sb > t[:, None, None]
    eq = sb == t[:, None, None]
    need = KEEP - jnp.sum(gt.astype(jnp.int32), axis=(1, 2))
    row = lax.broadcasted_iota(jnp.int32, (H, W), 0)
    col = lax.broadcasted_iota(jnp.int32, (H, W), 1)
    nidx = (row * W + col)[None]

    def istep(i, lo):
        cand = lo + ((1 << 13) >> i)
        cnt = jnp.sum((eq & (nidx <= cand[:, None, None])).astype(jnp.int32),
                      axis=(1, 2))
        return jnp.where(cnt < need, cand, lo)

    lo = lax.fori_loop(0, 14, istep, jnp.full((B,), -1, jnp.int32))

    tvec = jnp.broadcast_to(t[:, None, None], (B, 1, LANES))
    cvec = jnp.broadcast_to((lo + 1)[:, None, None], (B, 1, LANES))
    ti_ref[...] = jnp.concatenate([tvec, cvec], axis=1)


def _sc_pool(scores, tinfo, xt2d, B, C, N):
    mesh = plsc.VectorSubcoreMesh(core_axis_name="c", subcore_axis_name="s")
    n_chunks = KEEP // ROWS_PER_GATHER
    cg = C // LANES

    @functools.partial(
        pl.kernel,
        mesh=mesh,
        out_type=jax.ShapeDtypeStruct((B, cg, LANES), jnp.float32),
        scratch_types=[
            pltpu.VMEM((N,), jnp.int32),
            pltpu.VMEM((2, LANES), jnp.int32),
            pltpu.VMEM((KEEP + 128,), jnp.int32),
            pltpu.VMEM((n_chunks, ROWS_PER_GATHER), jnp.int32),
            pltpu.VMEM((ROWS_PER_GATHER, 128), jnp.float32),
            pltpu.VMEM((ROWS_PER_GATHER, 128), jnp.float32),
            pltpu.VMEM((cg, LANES), jnp.float32),
            pltpu.SemaphoreType.DMA,
            pltpu.SemaphoreType.DMA,
        ],
    )
    def body(scores_hbm, tinfo_hbm, xt_hbm, out_hbm,
             sc_v, ti_v, idxf_v, idx2_v, bufa_v, bufb_v, outrow_v,
             sema, semb):
        b = lax.axis_index("s") * SC_NC + lax.axis_index("c")
        pltpu.sync_copy(scores_hbm.at[b], sc_v)
        pltpu.sync_copy(tinfo_hbm.at[b], ti_v)
        tspl = ti_v[0]
        cutspl = ti_v[1]
        lane = lax.iota(jnp.int32, LANES)
        base = b * N

        def take16(v, idx):
            dnums = lax.GatherDimensionNumbers(
                offset_dims=(), collapsed_slice_dims=(0,),
                start_index_map=(0,))
            return lax.gather(
                v, idx[:, None], dnums, (1,),
                mode=lax.GatherScatterMode.PROMISE_IN_BOUNDS)

        ones = jnp.full((LANES,), 1, jnp.int32)
        zeros_i = jnp.zeros((LANES,), jnp.int32)
        def cstep(i, off):
            v = sc_v[pl.ds(i * LANES, LANES)]
            idxs = lane + i * LANES
            m = (v > tspl) | ((v == tspl) & (idxs <= cutspl))
            mi = jnp.where(m, ones, zeros_i)
            csum = mi
            for sh in (1, 2, 4, 8):
                shifted = take16(csum, jnp.maximum(lane - sh, 0))
                csum = csum + jnp.where(lane >= sh, shifted, zeros_i)
            tgt = lane + 1
            pos = zeros_i
            for step in (8, 4, 2, 1):
                cand = pos + step
                cval = take16(csum, cand - 1)
                pos = jnp.where(cval < tgt, cand, pos)
            vals = take16(idxs, pos) + base
            idxf_v[pl.ds(off, LANES)] = vals
            return off + csum[LANES - 1]

        lax.fori_loop(0, N // LANES, cstep, jnp.int32(0))

        for r in range(n_chunks):
            for k in range(ROWS_PER_GATHER // LANES):
                idx2_v[r, pl.ds(k * LANES, LANES)] = (
                    idxf_v[pl.ds(r * ROWS_PER_GATHER + k * LANES, LANES)])

        zeros = jnp.zeros((LANES,), jnp.float32)

        def reduce_chunk(buf, acc):
            def rstep(j, a):
                return tuple(a[k] + buf[j, pl.ds(k * LANES, LANES)]
                             for k in range(cg))

            return lax.fori_loop(0, ROWS_PER_GATHER, rstep, acc)

        bufs = (bufa_v, bufb_v)
        sems = (sema, semb)
        acc = (zeros,) * cg
        cps = [pltpu.async_copy(xt_hbm.at[idx2_v.at[0]], bufa_v, sema)]
        for q in range(n_chunks):
            if q + 1 < n_chunks:
                cps.append(pltpu.async_copy(
                    xt_hbm.at[idx2_v.at[q + 1]],
                    bufs[(q + 1) % 2], sems[(q + 1) % 2]))
            cps[q].wait()
            acc = reduce_chunk(bufs[q % 2], acc)
        for k in range(cg):
            outrow_v[k] = acc[k] * (1.0 / KEEP)
        pltpu.sync_copy(outrow_v, out_hbm.at[b])

    return body(scores, tinfo, xt2d)


def kernel(x):
    B, C, H, W = x.shape
    N = H * W
    sbits, xt = pl.pallas_call(
        _tc_stream_body,
        grid=(B // G,),
        in_specs=[pl.BlockSpec((G, C, H, W), lambda b: (b, 0, 0, 0))],
        out_specs=[
            pl.BlockSpec((G, H, W), lambda b: (b, 0, 0)),
            pl.BlockSpec((G, H, W, 128), lambda b: (b, 0, 0, 0)),
        ],
        out_shape=[
            jax.ShapeDtypeStruct((B, H, W), jnp.int32),
            jax.ShapeDtypeStruct((B, H, W, 128), jnp.float32),
        ],
    )(x)
    tinfo = pl.pallas_call(
        _tc_search_body,
        grid=(1,),
        in_specs=[pl.BlockSpec((B, H, W), lambda _: (0, 0, 0))],
        out_specs=pl.BlockSpec((B, 2, LANES), lambda _: (0, 0, 0)),
        out_shape=jax.ShapeDtypeStruct((B, 2, LANES), jnp.int32),
    )(sbits)
    scores = sbits.reshape(B, N)
    xt2d = xt.reshape(B * N, 128)
    out = _sc_pool(scores, tinfo, xt2d, B, C, N)
    return out.reshape(B, C)
